# baseline (speedup 1.0000x reference)
import functools
import numpy as np
import jax
import jax.numpy as jnp
from jax import lax
from jax.experimental import pallas as pl
from jax.experimental.pallas import tpu as pltpu
from jax.experimental.pallas import tpu_sc as plsc

VOCAB = 100000
CUTS = (0, 2000, 10000, 50000, 100000)
CUT1, CUT2, CUT3 = 2000, 10000, 50000
H = 768
XW = H + 128
LPAD = 2048
TT = 256
NTT = LPAD // TT
VT = 1024
NVT = (VOCAB + VT - 1) // VT
NSTEPS = NVT * NTT
LOG2E = 1.4426950408889634
LN2 = 0.6931471805599453

_c_lo = np.array([int(np.searchsorted(CUTS, v * VT, 'right') - 1)
                  for v in range(NVT)], np.int32)
_c_hi = np.array([int(np.searchsorted(CUTS, min((v + 1) * VT, VOCAB) - 1,
                                      'right') - 1)
                  for v in range(NVT)], np.int32)
_pure = (_c_lo == _c_hi).astype(np.int32)
if VOCAB % VT != 0:
    _pure[-1] = 0
_straddle = tuple(int(t) for t in np.where(_pure == 0)[0])
_v_of_j = np.repeat(np.arange(NVT, dtype=np.int32), NTT)
_k_of_j = np.tile(np.arange(NTT, dtype=np.int32), NVT)

_SC_NW = 32
_BPW = LPAD // _SC_NW


def _cluster_of(v):
    one = jnp.int32(1)
    return ((v >= CUT1) * one + (v >= CUT2) * one + (v >= CUT3) * one)


def _schedule(yf):
    n = yf.shape[0]
    cl = _cluster_of(yf)
    oh = (cl[:, None] == jnp.arange(4, dtype=jnp.int32)[None, :])
    pref = jnp.cumsum(oh.astype(jnp.int32), axis=0)
    counts = pref[-1]
    offs = jnp.concatenate([jnp.zeros((1,), jnp.int32),
                            jnp.cumsum(counts).astype(jnp.int32)])
    rank = jnp.take_along_axis(pref, cl[:, None], axis=1)[:, 0] - 1
    pos = (offs[cl] + rank).astype(jnp.int32)
    pos_pad = jnp.concatenate(
        [pos, jnp.full((LPAD - n,), LPAD - 1, jnp.int32)])

    start = offs[_c_lo]
    end = offs[_c_hi + 1]
    tlo = (start // TT).astype(jnp.int32)
    cnt = jnp.where(end > start,
                    (end + TT - 1) // TT - start // TT, 0).astype(jnp.int32)
    tt = jnp.clip(tlo[_v_of_j] + _k_of_j, 0, NTT - 1).astype(jnp.int32)
    valid = (_k_of_j < cnt[_v_of_j]).astype(jnp.int32)
    return pos_pad, tt, valid


def _grouped_body(tt_ref, valid_ref, x_ref, w_ref, b_ref, cw_ref, cb_ref,
                  out_ref, s_acc, t_acc, cll):
    j = pl.program_id(0)
    v = j // NTT

    @pl.when(j == 0)
    def _init():
        s_acc[:] = jnp.zeros_like(s_acc)
        t_acc[:] = jnp.zeros_like(t_acc)
        clg = jnp.dot(x_ref[:, :H], cw_ref[:],
                      preferred_element_type=jnp.float32) * LN2 + cb_ref[:]
        m = jnp.max(clg, axis=1, keepdims=True)
        lse = m + jnp.log(jnp.sum(jnp.exp(clg - m), axis=1, keepdims=True))
        ccol = jax.lax.broadcasted_iota(jnp.int32, (1, clg.shape[1]), 1)
        tok_cl = _cluster_of(x_ref[:, H:H + 1])
        cll[:] = jnp.sum(jnp.where(ccol == tok_cl, clg - lse, 0.0),
                         axis=1, keepdims=True)

    @pl.when(valid_ref[j] != 0)
    def _item():
        r0 = tt_ref[j] * TT
        xt = x_ref[pl.ds(r0, TT), :H]
        l = jnp.dot(xt, w_ref[:],
                    preferred_element_type=jnp.float32) + b_ref[:]
        col = v * VT + jax.lax.broadcasted_iota(jnp.int32, (1, VT), 1)
        ytf = x_ref[pl.ds(r0, TT), H:H + 1]
        tok_cl = _cluster_of(ytf)
        e = jnp.exp2(l)
        is_pure = jnp.bool_(True)
        for _s in _straddle:
            is_pure = jnp.logical_and(is_pure, v != _s)
        cid = _cluster_of(v * VT)

        @pl.when(is_pure)
        def _fast():
            ssum = jnp.sum(e, axis=1, keepdims=True)
            s_acc[pl.ds(r0, TT), :] += jnp.where(tok_cl == cid, ssum, 0.0)

        @pl.when(jnp.logical_not(is_pure))
        def _slow():
            col_cl = jnp.where(col < VOCAB, _cluster_of(col), -1)
            s_acc[pl.ds(r0, TT), :] += jnp.sum(
                jnp.where(col_cl == tok_cl, e, 0.0),
                axis=1, keepdims=True)

        t_acc[pl.ds(r0, TT), :] += jnp.sum(
            jnp.where(col.astype(jnp.float32) == ytf, l, 0.0),
            axis=1, keepdims=True)

    @pl.when(j == NSTEPS - 1)
    def _finish():
        nll = -(cll[:] + LN2 * (t_acc[:] - jnp.log2(s_acc[:])))
        out_ref[:] = jnp.broadcast_to(nll, (LPAD, 128))


def _tc_grouped(x_s, W, b2, cW, cb, tt, valid):
    grid_spec = pltpu.PrefetchScalarGridSpec(
        num_scalar_prefetch=2,
        grid=(NSTEPS,),
        in_specs=[
            pl.BlockSpec((LPAD, XW), lambda j, *s: (0, 0)),
            pl.BlockSpec((H, VT), lambda j, *s: (0, j // NTT)),
            pl.BlockSpec((1, VT), lambda j, *s: (0, j // NTT)),
            pl.BlockSpec(cW.shape, lambda j, *s: (0, 0)),
            pl.BlockSpec(cb.shape, lambda j, *s: (0, 0)),
        ],
        out_specs=pl.BlockSpec((LPAD, 128), lambda j, *s: (0, 0)),
        scratch_shapes=[
            pltpu.VMEM((LPAD, 1), jnp.float32),
            pltpu.VMEM((LPAD, 1), jnp.float32),
            pltpu.VMEM((LPAD, 1), jnp.float32),
        ],
    )
    return pl.pallas_call(
        _grouped_body,
        grid_spec=grid_spec,
        out_shape=jax.ShapeDtypeStruct((LPAD, 128), jnp.float32),
        compiler_params=pltpu.CompilerParams(
            dimension_semantics=("arbitrary",)),
    )(tt, valid, x_s, W, b2, cW, cb)


def _sc_scatter_x(x_aug, pos_pad):
    mesh = plsc.VectorSubcoreMesh(core_axis_name="c", subcore_axis_name="s")

    @functools.partial(
        pl.kernel, mesh=mesh,
        out_type=jax.ShapeDtypeStruct((LPAD, XW), jnp.float32),
        scratch_types=[
            pltpu.VMEM((_BPW,), jnp.int32),
            pltpu.VMEM((_BPW, XW), jnp.float32),
            pltpu.SemaphoreType.DMA,
        ],
    )
    def k(x_hbm, idx_hbm, out_hbm, idx_v, rows_v, sem):
        wid = lax.axis_index("s") * 2 + lax.axis_index("c")
        b0 = wid * _BPW
        pltpu.sync_copy(idx_hbm.at[pl.ds(b0, _BPW)], idx_v)
        pltpu.sync_copy(x_hbm.at[pl.ds(b0, _BPW)], rows_v)
        pltpu.async_copy(rows_v, out_hbm.at[idx_v], sem).wait()

    return k(x_aug, pos_pad)


def _sc_gather_out(src, pos_pad):
    mesh = plsc.VectorSubcoreMesh(core_axis_name="c", subcore_axis_name="s")

    @functools.partial(
        pl.kernel, mesh=mesh,
        out_type=jax.ShapeDtypeStruct((LPAD, 128), jnp.float32),
        scratch_types=[
            pltpu.VMEM((_BPW,), jnp.int32),
            pltpu.VMEM((_BPW, 128), jnp.float32),
            pltpu.SemaphoreType.DMA,
        ],
    )
    def k(src_hbm, idx_hbm, out_hbm, idx_v, rows_v, sem):
        wid = lax.axis_index("s") * 2 + lax.axis_index("c")
        b0 = wid * _BPW
        pltpu.sync_copy(idx_hbm.at[pl.ds(b0, _BPW)], idx_v)
        pltpu.async_copy(src_hbm.at[idx_v], rows_v, sem).wait()
        pltpu.sync_copy(rows_v, out_hbm.at[pl.ds(b0, _BPW)])

    return k(src, pos_pad)


def kernel(x, y, cluster_W, cluster_b, W, b):
    x = x[:, :-1]
    bsz, l, h = x.shape
    xf = x.reshape(bsz * l, h)
    yf = y.reshape(-1)
    n = xf.shape[0]
    x_aug = jnp.concatenate(
        [jnp.pad(xf, ((0, LPAD - n), (0, 0))) * jnp.float32(LOG2E),
         jnp.pad(yf.astype(jnp.float32)[:, None],
                 ((0, LPAD - n), (0, 128 - 1)))], axis=1)
    b2 = b * jnp.float32(LOG2E)

    pos_pad, tt, valid = _schedule(yf)
    x_s = _sc_scatter_x(x_aug, pos_pad)
    nll_s = _tc_grouped(x_s, W, b2, cluster_W, cluster_b, tt, valid)
    nll = _sc_gather_out(nll_s, pos_pad)
    return nll[:n, 0]
